# baseline (speedup 1.0000x reference)
import jax
import jax.numpy as jnp
import numpy as np
from jax.experimental import pallas as pl
from jax.experimental.pallas import tpu as pltpu

_N = 8
_EI = np.array([[3, 0, 3, 1, 3, 2, 3, 7, 7, 4, 7, 5, 7, 6, 0, 1, 1, 6, 6, 4, 4, 5, 5, 2, 2, 0],
                [0, 3, 1, 3, 2, 3, 7, 3, 4, 7, 5, 7, 6, 7, 1, 0, 6, 1, 4, 6, 5, 4, 2, 5, 0, 2]],
               dtype=np.int64)
_src = np.concatenate([_EI[0], np.arange(_N, dtype=np.int64)])
_dst = np.concatenate([_EI[1], np.arange(_N, dtype=np.int64)])
_deg = np.zeros(_N, dtype=np.float32)
np.add.at(_deg, _dst, 1.0)
_norm = (_deg[_src] ** -0.5) * (_deg[_dst] ** -0.5)
_A = np.zeros((_N, _N), dtype=np.float32)
np.add.at(_A, (_dst, _src), _norm)

_BB = 512


def _nodemix(h, a):
    acc = None
    for m in range(_N):
        term = a[None, :, m:m + 1] * h[:, m:m + 1, :]
        acc = term if acc is None else acc + term
    return acc


def _body(x_ref, a_ref, w0_ref, b0_ref, w1_ref, b1_ref, w2_ref, b2_ref,
          r0_ref, rb0_ref, r1_ref, rb1_ref, r2t_ref, rb2_ref, o_ref):
    f32 = jnp.float32
    bB = x_ref.shape[0]
    a = a_ref[...]
    x = x_ref[...]
    h = jnp.dot(x.reshape(bB * _N, 128), w0_ref[...], preferred_element_type=f32)
    h = _nodemix(h.reshape(bB, _N, 64), a)
    h = jnp.maximum(h + b0_ref[...][None], 0.0)

    h = _nodemix(h, a).reshape(bB * _N, 64)
    h = jnp.dot(h, w1_ref[...], preferred_element_type=f32)
    h = jnp.maximum(h + b1_ref[...], 0.0)

    h = _nodemix(h.reshape(bB, _N, 96), a).reshape(bB * _N, 96)
    h = jnp.dot(h, w2_ref[...], preferred_element_type=f32)
    h = jnp.maximum(h + b2_ref[...], 0.0)

    y = jnp.maximum(jnp.dot(h, r0_ref[...], preferred_element_type=f32) + rb0_ref[...], 0.0)
    y = jnp.maximum(jnp.dot(y, r1_ref[...], preferred_element_type=f32) + rb1_ref[...], 0.0)
    s = jnp.sum(y.reshape(bB, _N, 32) * r2t_ref[...][None], axis=2) + rb2_ref[0, 0]
    o_ref[...] = jnp.max(s, axis=1, keepdims=True)


def kernel(x, W0, b0, W1, b1, W2, b2, R0, rb0, R1, rb1, R2, rb2):
    B = x.shape[0]
    bB = _BB
    grid = (B // bB,)

    full = lambda shape: pl.BlockSpec(shape, lambda i: (0,) * len(shape))
    out = pl.pallas_call(
        _body,
        grid=grid,
        in_specs=[
            pl.BlockSpec((bB, _N, 128), lambda i: (i, 0, 0)),
            full((_N, _N)),
            full((128, 64)), full((1, 64)),
            full((64, 96)), full((1, 96)),
            full((96, 128)), full((1, 128)),
            full((128, 64)), full((1, 64)),
            full((64, 32)), full((1, 32)),
            full((1, 32)), full((1, 1)),
        ],
        out_specs=pl.BlockSpec((bB, 1), lambda i: (i, 0)),
        out_shape=jax.ShapeDtypeStruct((B, 1), jnp.float32),
        compiler_params=pltpu.CompilerParams(
            dimension_semantics=("parallel",),
        ),
    )(x, jnp.asarray(_A), W0, b0.reshape(1, 64), W1, b1.reshape(1, 96), W2, b2.reshape(1, 128),
      R0, rb0.reshape(1, 64), R1, rb1.reshape(1, 32),
      R2.reshape(1, 32), rb2.reshape(1, 1))
    return out


# speedup vs baseline: 1.5721x; 1.5721x over previous
import jax
import jax.numpy as jnp
import numpy as np
from jax.experimental import pallas as pl
from jax.experimental.pallas import tpu as pltpu

_N = 8
_EI = np.array([[3, 0, 3, 1, 3, 2, 3, 7, 7, 4, 7, 5, 7, 6, 0, 1, 1, 6, 6, 4, 4, 5, 5, 2, 2, 0],
                [0, 3, 1, 3, 2, 3, 7, 3, 4, 7, 5, 7, 6, 7, 1, 0, 6, 1, 4, 6, 5, 4, 2, 5, 0, 2]],
               dtype=np.int64)
_src = np.concatenate([_EI[0], np.arange(_N, dtype=np.int64)])
_dst = np.concatenate([_EI[1], np.arange(_N, dtype=np.int64)])
_deg = np.zeros(_N, dtype=np.float32)
np.add.at(_deg, _dst, 1.0)
_M01 = np.zeros((_N, _N), dtype=np.float32)
_M01[_dst, _src] = 1.0
_SVEC = (_deg ** -0.5).astype(np.float32)

_CHUNK = 128
_BD01 = np.kron(np.eye(_CHUNK // _N, dtype=np.float32), _M01)
_M2 = np.hstack([_BD01, _BD01]).astype(np.float32)

_BB = 512


def _body(x_ref, m2_ref, s_ref, w0_ref, b0_ref, w1_ref, b1_ref, w2_ref, b2_ref,
          r0_ref, rb0_ref, r1_ref, rb1_ref, r2t_ref, rb2_ref, o_ref):
    f32 = jnp.float32
    bf16 = jnp.bfloat16
    bB = x_ref.shape[0]
    R = bB * _N
    m2 = m2_ref[...].astype(bf16)
    s3 = s_ref[...][:, :, None]

    def mix(z, C):
        u = (z.reshape(bB, _N, C) * s3).reshape(R, C)
        uhi = u.astype(bf16)
        ulo = (u - uhi.astype(f32)).astype(bf16)
        outs = []
        for g in range(R // _CHUNK):
            lo, hi = g * _CHUNK, (g + 1) * _CHUNK
            U = jnp.concatenate([uhi[lo:hi], ulo[lo:hi]], axis=0)
            outs.append(jnp.dot(m2, U, preferred_element_type=f32))
        v = jnp.concatenate(outs, axis=0)
        return (v.reshape(bB, _N, C) * s3).reshape(R, C)

    _dot = lambda a, b: jnp.dot(a, b, preferred_element_type=f32)

    x = x_ref[...]
    h = _dot(x.reshape(R, 128), w0_ref[...])
    h = jnp.maximum(mix(h, 64) + b0_ref[...], 0.0)
    h = _dot(mix(h, 64), w1_ref[...])
    h = jnp.maximum(h + b1_ref[...], 0.0)
    h = _dot(mix(h, 96), w2_ref[...])
    h = jnp.maximum(h + b2_ref[...], 0.0)

    y = jnp.maximum(_dot(h, r0_ref[...]) + rb0_ref[...], 0.0)
    y = jnp.maximum(_dot(y, r1_ref[...]) + rb1_ref[...], 0.0)
    yb = y.astype(bf16).astype(f32)
    r2b = r2t_ref[...].astype(bf16).astype(f32)
    s_out = jnp.sum(yb.reshape(bB, _N, 32) * r2b[None], axis=2) + rb2_ref[0, 0]
    o_ref[...] = jnp.max(s_out, axis=1, keepdims=True)


def kernel(x, W0, b0, W1, b1, W2, b2, R0, rb0, R1, rb1, R2, rb2):
    B = x.shape[0]
    bB = _BB
    grid = (B // bB,)

    full = lambda shape: pl.BlockSpec(shape, lambda i: (0,) * len(shape))
    out = pl.pallas_call(
        _body,
        grid=grid,
        in_specs=[
            pl.BlockSpec((bB, _N, 128), lambda i: (i, 0, 0)),
            full((_CHUNK, 2 * _CHUNK)), full((1, _N)),
            full((128, 64)), full((1, 64)),
            full((64, 96)), full((1, 96)),
            full((96, 128)), full((1, 128)),
            full((128, 64)), full((1, 64)),
            full((64, 32)), full((1, 32)),
            full((1, 32)), full((1, 1)),
        ],
        out_specs=pl.BlockSpec((bB, 1), lambda i: (i, 0)),
        out_shape=jax.ShapeDtypeStruct((B, 1), jnp.float32),
        compiler_params=pltpu.CompilerParams(
            dimension_semantics=("parallel",),
        ),
    )(x, jnp.asarray(_M2), jnp.asarray(_SVEC.reshape(1, _N)),
      W0, b0.reshape(1, 64), W1, b1.reshape(1, 96), W2, b2.reshape(1, 128),
      R0, rb0.reshape(1, 64), R1, rb1.reshape(1, 32),
      R2.reshape(1, 32), rb2.reshape(1, 1))
    return out


# speedup vs baseline: 1.6040x; 1.0203x over previous
import jax
import jax.numpy as jnp
import numpy as np
from jax.experimental import pallas as pl
from jax.experimental.pallas import tpu as pltpu

_N = 8
_EI = np.array([[3, 0, 3, 1, 3, 2, 3, 7, 7, 4, 7, 5, 7, 6, 0, 1, 1, 6, 6, 4, 4, 5, 5, 2, 2, 0],
                [0, 3, 1, 3, 2, 3, 7, 3, 4, 7, 5, 7, 6, 7, 1, 0, 6, 1, 4, 6, 5, 4, 2, 5, 0, 2]],
               dtype=np.int64)
_src = np.concatenate([_EI[0], np.arange(_N, dtype=np.int64)])
_dst = np.concatenate([_EI[1], np.arange(_N, dtype=np.int64)])
_deg = np.zeros(_N, dtype=np.float32)
np.add.at(_deg, _dst, 1.0)
_M01 = np.zeros((_N, _N), dtype=np.float32)
_M01[_dst, _src] = 1.0
_SVEC = (_deg ** -0.5).astype(np.float32)

_CHUNK = 128
_BD01 = np.kron(np.eye(_CHUNK // _N, dtype=np.float32), _M01)

_BB = 512


def _body(x_ref, m_ref, s_ref, w0_ref, b0_ref, w1_ref, b1_ref, w2_ref, b2_ref,
          r0_ref, rb0_ref, r1_ref, rb1_ref, r2t_ref, rb2_ref, o_ref):
    f32 = jnp.float32
    bf16 = jnp.bfloat16
    bB = x_ref.shape[0]
    R = bB * _N
    m01 = m_ref[...]
    s3 = s_ref[...][:, :, None]

    def mix(z, C):
        u = (z.reshape(bB, _N, C) * s3).reshape(R, C)
        uhi = u.astype(bf16)
        ulo = (u - uhi.astype(f32)).astype(bf16)
        outs = []
        for g in range(R // _CHUNK):
            lo, hi = g * _CHUNK, (g + 1) * _CHUNK
            outs.append(jnp.dot(m01, uhi[lo:hi], preferred_element_type=f32)
                        + jnp.dot(m01, ulo[lo:hi], preferred_element_type=f32))
        v = jnp.concatenate(outs, axis=0)
        return (v.reshape(bB, _N, C) * s3).reshape(R, C)

    _dot = lambda a, b: jnp.dot(a, b, preferred_element_type=f32)

    x = x_ref[...]
    h = _dot(x.reshape(R, 128), w0_ref[...])
    h = jnp.maximum(mix(h, 64) + b0_ref[...], 0.0)
    h = _dot(h, w1_ref[...])
    h = jnp.maximum(mix(h, 96) + b1_ref[...], 0.0)
    h = _dot(h, w2_ref[...])
    h = jnp.maximum(mix(h, 128) + b2_ref[...], 0.0)

    y = jnp.maximum(_dot(h, r0_ref[...]) + rb0_ref[...], 0.0)
    y = jnp.maximum(_dot(y, r1_ref[...]) + rb1_ref[...], 0.0)
    yb = y.astype(bf16).astype(f32)
    r2b = r2t_ref[...].astype(bf16).astype(f32)
    s_out = jnp.sum(yb.reshape(bB, _N, 32) * r2b[None], axis=2) + rb2_ref[0, 0]
    o_ref[...] = jnp.max(s_out, axis=1, keepdims=True)


def kernel(x, W0, b0, W1, b1, W2, b2, R0, rb0, R1, rb1, R2, rb2):
    B = x.shape[0]
    bB = _BB
    grid = (B // bB,)

    full = lambda shape: pl.BlockSpec(shape, lambda i: (0,) * len(shape))
    out = pl.pallas_call(
        _body,
        grid=grid,
        in_specs=[
            pl.BlockSpec((bB, _N, 128), lambda i: (i, 0, 0)),
            full((_CHUNK, _CHUNK)), full((1, _N)),
            full((128, 64)), full((1, 64)),
            full((64, 96)), full((1, 96)),
            full((96, 128)), full((1, 128)),
            full((128, 64)), full((1, 64)),
            full((64, 32)), full((1, 32)),
            full((1, 32)), full((1, 1)),
        ],
        out_specs=pl.BlockSpec((bB, 1), lambda i: (i, 0)),
        out_shape=jax.ShapeDtypeStruct((B, 1), jnp.float32),
        compiler_params=pltpu.CompilerParams(
            dimension_semantics=("parallel",),
        ),
    )(x, jnp.asarray(_BD01, dtype=jnp.bfloat16), jnp.asarray(_SVEC.reshape(1, _N)),
      W0, b0.reshape(1, 64), W1, b1.reshape(1, 96), W2, b2.reshape(1, 128),
      R0, rb0.reshape(1, 64), R1, rb1.reshape(1, 32),
      R2.reshape(1, 32), rb2.reshape(1, 1))
    return out


# speedup vs baseline: 1.6593x; 1.0345x over previous
import jax
import jax.numpy as jnp
import numpy as np
from jax.experimental import pallas as pl
from jax.experimental.pallas import tpu as pltpu

_N = 8
_EI = np.array([[3, 0, 3, 1, 3, 2, 3, 7, 7, 4, 7, 5, 7, 6, 0, 1, 1, 6, 6, 4, 4, 5, 5, 2, 2, 0],
                [0, 3, 1, 3, 2, 3, 7, 3, 4, 7, 5, 7, 6, 7, 1, 0, 6, 1, 4, 6, 5, 4, 2, 5, 0, 2]],
               dtype=np.int64)
_src = np.concatenate([_EI[0], np.arange(_N, dtype=np.int64)])
_dst = np.concatenate([_EI[1], np.arange(_N, dtype=np.int64)])
_deg = np.zeros(_N, dtype=np.float32)
np.add.at(_deg, _dst, 1.0)
_M01 = np.zeros((_N, _N), dtype=np.float32)
_M01[_dst, _src] = 1.0
_SVEC = (_deg ** -0.5).astype(np.float32)

_CHUNK = 128
_BD01 = np.kron(np.eye(_CHUNK // _N, dtype=np.float32), _M01)

_BB = 1024


def _body(x_ref, m_ref, s_ref, w0_ref, b0_ref, w1_ref, b1_ref, w2_ref, b2_ref,
          r0_ref, rb0_ref, r1_ref, rb1_ref, r2t_ref, rb2_ref, o_ref):
    f32 = jnp.float32
    bf16 = jnp.bfloat16
    bB = x_ref.shape[0]
    R = bB * _N
    m01 = m_ref[...]
    s3 = s_ref[...][:, :, None]

    def mix(z, C):
        u = (z.reshape(bB, _N, C) * s3).reshape(R, C)
        uhi = u.astype(bf16)
        ulo = (u - uhi.astype(f32)).astype(bf16)
        outs = []
        for g in range(R // _CHUNK):
            lo, hi = g * _CHUNK, (g + 1) * _CHUNK
            outs.append(jnp.dot(m01, uhi[lo:hi], preferred_element_type=f32)
                        + jnp.dot(m01, ulo[lo:hi], preferred_element_type=f32))
        v = jnp.concatenate(outs, axis=0)
        return (v.reshape(bB, _N, C) * s3).reshape(R, C)

    _dot = lambda a, b: jnp.dot(a, b, preferred_element_type=f32)

    x = x_ref[...]
    h = _dot(x.reshape(R, 128), w0_ref[...])
    h = jnp.maximum(mix(h, 64) + b0_ref[...], 0.0)
    h = _dot(h, w1_ref[...])
    h = jnp.maximum(mix(h, 96) + b1_ref[...], 0.0)
    h = _dot(h, w2_ref[...])
    h = jnp.maximum(mix(h, 128) + b2_ref[...], 0.0)

    y = jnp.maximum(_dot(h, r0_ref[...]) + rb0_ref[...], 0.0)
    y = jnp.maximum(_dot(y, r1_ref[...]) + rb1_ref[...], 0.0)
    yb = y.astype(bf16).astype(f32)
    r2b = r2t_ref[...].astype(bf16).astype(f32)
    s_out = jnp.sum(yb.reshape(bB, _N, 32) * r2b[None], axis=2) + rb2_ref[0, 0]
    o_ref[...] = jnp.max(s_out, axis=1, keepdims=True)


def kernel(x, W0, b0, W1, b1, W2, b2, R0, rb0, R1, rb1, R2, rb2):
    B = x.shape[0]
    bB = _BB
    grid = (B // bB,)

    full = lambda shape: pl.BlockSpec(shape, lambda i: (0,) * len(shape))
    out = pl.pallas_call(
        _body,
        grid=grid,
        in_specs=[
            pl.BlockSpec((bB, _N, 128), lambda i: (i, 0, 0)),
            full((_CHUNK, _CHUNK)), full((1, _N)),
            full((128, 64)), full((1, 64)),
            full((64, 96)), full((1, 96)),
            full((96, 128)), full((1, 128)),
            full((128, 64)), full((1, 64)),
            full((64, 32)), full((1, 32)),
            full((1, 32)), full((1, 1)),
        ],
        out_specs=pl.BlockSpec((bB, 1), lambda i: (i, 0)),
        out_shape=jax.ShapeDtypeStruct((B, 1), jnp.float32),
        compiler_params=pltpu.CompilerParams(
            dimension_semantics=("parallel",),
        ),
    )(x, jnp.asarray(_BD01, dtype=jnp.bfloat16), jnp.asarray(_SVEC.reshape(1, _N)),
      W0, b0.reshape(1, 64), W1, b1.reshape(1, 96), W2, b2.reshape(1, 128),
      R0, rb0.reshape(1, 64), R1, rb1.reshape(1, 32),
      R2.reshape(1, 32), rb2.reshape(1, 1))
    return out


# speedup vs baseline: 1.6666x; 1.0044x over previous
import jax
import jax.numpy as jnp
import numpy as np
from jax.experimental import pallas as pl
from jax.experimental.pallas import tpu as pltpu

_N = 8
_EI = np.array([[3, 0, 3, 1, 3, 2, 3, 7, 7, 4, 7, 5, 7, 6, 0, 1, 1, 6, 6, 4, 4, 5, 5, 2, 2, 0],
                [0, 3, 1, 3, 2, 3, 7, 3, 4, 7, 5, 7, 6, 7, 1, 0, 6, 1, 4, 6, 5, 4, 2, 5, 0, 2]],
               dtype=np.int64)
_src = np.concatenate([_EI[0], np.arange(_N, dtype=np.int64)])
_dst = np.concatenate([_EI[1], np.arange(_N, dtype=np.int64)])
_deg = np.zeros(_N, dtype=np.float32)
np.add.at(_deg, _dst, 1.0)
_M01 = np.zeros((_N, _N), dtype=np.float32)
_M01[_dst, _src] = 1.0
_SVEC = (_deg ** -0.5).astype(np.float32)

_CHUNK = 128
_BD01 = np.kron(np.eye(_CHUNK // _N, dtype=np.float32), _M01)

_BB = 2048


def _body(x_ref, m_ref, s_ref, w0_ref, b0_ref, w1_ref, b1_ref, w2_ref, b2_ref,
          r0_ref, rb0_ref, r1_ref, rb1_ref, r2t_ref, rb2_ref, o_ref):
    f32 = jnp.float32
    bf16 = jnp.bfloat16
    bB = x_ref.shape[0]
    R = bB * _N
    m01 = m_ref[...]
    s3 = s_ref[...][:, :, None]

    def mix(z, C):
        u = (z.reshape(bB, _N, C) * s3).reshape(R, C)
        uhi = u.astype(bf16)
        ulo = (u - uhi.astype(f32)).astype(bf16)
        outs = []
        for g in range(R // _CHUNK):
            lo, hi = g * _CHUNK, (g + 1) * _CHUNK
            outs.append(jnp.dot(m01, uhi[lo:hi], preferred_element_type=f32)
                        + jnp.dot(m01, ulo[lo:hi], preferred_element_type=f32))
        v = jnp.concatenate(outs, axis=0)
        return (v.reshape(bB, _N, C) * s3).reshape(R, C)

    _dot = lambda a, b: jnp.dot(a, b, preferred_element_type=f32)

    x = x_ref[...]
    h = _dot(x.reshape(R, 128), w0_ref[...])
    h = jnp.maximum(mix(h, 64) + b0_ref[...], 0.0)
    h = _dot(h, w1_ref[...])
    h = jnp.maximum(mix(h, 96) + b1_ref[...], 0.0)
    h = _dot(h, w2_ref[...])
    h = jnp.maximum(mix(h, 128) + b2_ref[...], 0.0)

    y = jnp.maximum(_dot(h, r0_ref[...]) + rb0_ref[...], 0.0)
    y = jnp.maximum(_dot(y, r1_ref[...]) + rb1_ref[...], 0.0)
    yb = y.astype(bf16).astype(f32)
    r2b = r2t_ref[...].astype(bf16).astype(f32)
    s_out = jnp.sum(yb.reshape(bB, _N, 32) * r2b[None], axis=2) + rb2_ref[0, 0]
    o_ref[...] = jnp.max(s_out, axis=1, keepdims=True)


def kernel(x, W0, b0, W1, b1, W2, b2, R0, rb0, R1, rb1, R2, rb2):
    B = x.shape[0]
    bB = _BB
    grid = (B // bB,)

    full = lambda shape: pl.BlockSpec(shape, lambda i: (0,) * len(shape))
    out = pl.pallas_call(
        _body,
        grid=grid,
        in_specs=[
            pl.BlockSpec((bB, _N, 128), lambda i: (i, 0, 0)),
            full((_CHUNK, _CHUNK)), full((1, _N)),
            full((128, 64)), full((1, 64)),
            full((64, 96)), full((1, 96)),
            full((96, 128)), full((1, 128)),
            full((128, 64)), full((1, 64)),
            full((64, 32)), full((1, 32)),
            full((1, 32)), full((1, 1)),
        ],
        out_specs=pl.BlockSpec((bB, 1), lambda i: (i, 0)),
        out_shape=jax.ShapeDtypeStruct((B, 1), jnp.float32),
        compiler_params=pltpu.CompilerParams(
            dimension_semantics=("parallel",),
        ),
    )(x, jnp.asarray(_BD01, dtype=jnp.bfloat16), jnp.asarray(_SVEC.reshape(1, _N)),
      W0, b0.reshape(1, 64), W1, b1.reshape(1, 96), W2, b2.reshape(1, 128),
      R0, rb0.reshape(1, 64), R1, rb1.reshape(1, 32),
      R2.reshape(1, 32), rb2.reshape(1, 1))
    return out
